# baseline (speedup 1.0000x reference)
import functools

import jax
import jax.numpy as jnp
from jax import lax
from jax.experimental import pallas as pl
from jax.experimental.pallas import tpu as pltpu
from jax.experimental.pallas import tpu_sc as plsc

_N = 10000
_E = 160000
_LANE = 128
_NSUB = 16
_NCORE = 2
_NW = _NCORE * _NSUB
_EPW = _E // _NW
_B = 50
_NBLK = _EPW // _B
_L = ((_NBLK - 2) // 3) * 3
_NPAD = 10240
_RPS = _NPAD // _NSUB
_P = 128
_NPIECE = _RPS // _P
_PRB = 40
_NPRB = _RPS // _PRB
_R = 1000
_GRID = _N // _R


def _make_sc_segsum(c_chunks):
    mesh = plsc.VectorSubcoreMesh(core_axis_name="c", subcore_axis_name="s")

    @functools.partial(
        pl.kernel,
        out_type=jax.ShapeDtypeStruct((_NCORE, c_chunks, _NPAD, _LANE), jnp.float32),
        mesh=mesh,
        scratch_types=[
            pltpu.VMEM((_NBLK, _B), jnp.int32),
            pltpu.VMEM((_NBLK, _B), jnp.int32),
            pltpu.VMEM((_B, _LANE), jnp.float32),
            pltpu.VMEM((_B, _LANE), jnp.float32),
            pltpu.VMEM((_B, _LANE), jnp.float32),
            pltpu.SemaphoreType.DMA,
            pltpu.SemaphoreType.DMA,
            pltpu.SemaphoreType.DMA,
            pltpu.SemaphoreType.DMA,
            pltpu.SemaphoreType.DMA,
            pltpu.SemaphoreType.DMA,
            pltpu.VMEM_SHARED((_NPAD, _LANE), jnp.float32),
        ],
    )
    def segsum(x_hbm, src_hbm, dst_hbm, z_hbm, out_hbm,
               src_v, dst_v, b0, b1, b2, g0, g1, g2, s0, s1, s2, acc_sh):
        core = lax.axis_index("c")
        sub = lax.axis_index("s")
        wid = core * _NSUB + sub
        bufs = (b0, b1, b2)
        gs = (g0, g1, g2)
        ss = (s0, s1, s2)
        pltpu.sync_copy(src_hbm.at[wid], src_v)
        pltpu.sync_copy(dst_hbm.at[wid], dst_v)

        def gather(c, k, i):
            return pltpu.make_async_copy(x_hbm.at[c].at[src_v.at[k]], bufs[i],
                                         gs[i])

        def scatter(k, i):
            return pltpu.async_copy(bufs[i], acc_sh.at[dst_v.at[k]], ss[i],
                                    add=True)

        def scatter_wait(k, i):
            pltpu.make_async_copy(bufs[i], acc_sh.at[dst_v.at[k]], ss[i]).wait()

        for c in range(c_chunks):
            for p in range(_NPIECE):
                pltpu.sync_copy(z_hbm, acc_sh.at[pl.ds(sub * _RPS + p * _P, _P)])
            plsc.subcore_barrier()

            for i in range(3):
                gather(c, i, i).start()

            @pl.loop(0, _L, step=3)
            def _(j):
                for i in range(3):
                    k = j + i
                    gather(c, k, i).wait()
                    ip = (i + 2) % 3
                    pltpu.sync_copy(bufs[i], acc_sh.at[dst_v.at[k]], add=True)
                    if i == 0:
                        @pl.when(j > 0)
                        def _():
                            gather(c, j + 2, ip).start()
                    else:
                        gather(c, k + 2, ip).start()

            for k in range(_L, _NBLK):
                i = k % 3
                gather(c, k, i).wait()
                pltpu.sync_copy(bufs[i], acc_sh.at[dst_v.at[k]], add=True)
                if _L + 2 <= k + 2 < _NBLK:
                    gather(c, k + 2, (k + 2) % 3).start()

            plsc.subcore_barrier()
            for p in range(_NPRB):
                r0 = sub * _RPS + p * _PRB
                pltpu.sync_copy(acc_sh.at[pl.ds(r0, _PRB)], b0.at[pl.ds(0, _PRB)])
                pltpu.sync_copy(b0.at[pl.ds(0, _PRB)], out_hbm.at[core, c, pl.ds(r0, _PRB)])
            plsc.subcore_barrier()

    return segsum


def _tc1_body(c_chunks, x_ref, a_ref, w_ref, b1_ref, h1_ref, st_ref):
    acc = jnp.zeros((_R, 512), jnp.float32)
    for c in range(c_chunks):
        xa = x_ref[c] + a_ref[0, c] + a_ref[1, c]
        acc += jnp.dot(xa, w_ref[c], preferred_element_type=jnp.float32)
    h1 = acc + b1_ref[...]
    h1_ref[...] = h1
    i = pl.program_id(0)

    @pl.when(i == 0)
    def _():
        st_ref[...] = jnp.zeros((8, 512), jnp.float32)

    st_ref[0:1, :] += jnp.sum(h1, axis=0, keepdims=True)
    st_ref[1:2, :] += jnp.sum(h1 * h1, axis=0, keepdims=True)


def _tc1(x_ch, agg2, w1c, b1, c_chunks):
    return pl.pallas_call(
        functools.partial(_tc1_body, c_chunks),
        grid=(_GRID,),
        in_specs=[
            pl.BlockSpec((c_chunks, _R, _LANE), lambda i: (0, i, 0)),
            pl.BlockSpec((_NCORE, c_chunks, _R, _LANE), lambda i: (0, 0, i, 0)),
            pl.BlockSpec((c_chunks, _LANE, 512), lambda i: (0, 0, 0)),
            pl.BlockSpec((1, 512), lambda i: (0, 0)),
        ],
        out_specs=[
            pl.BlockSpec((_R, 512), lambda i: (i, 0)),
            pl.BlockSpec((8, 512), lambda i: (0, 0)),
        ],
        out_shape=[
            jax.ShapeDtypeStruct((_N, 512), jnp.float32),
            jax.ShapeDtypeStruct((8, 512), jnp.float32),
        ],
    )(x_ch, agg2, w1c, b1)


def _tc2_body(chunked, h1_ref, st_ref, g_ref, be_ref, w_ref, b2_ref, out_ref):
    inv_n = 1.0 / _N
    mu = st_ref[0:1, :] * inv_n
    ex2 = st_ref[1:2, :] * inv_n
    var = ex2 - mu * mu
    a = g_ref[...] * lax.rsqrt(var + 1e-5)
    cc = be_ref[...] - mu * a
    t = h1_ref[...] * a + cc
    t = jnp.where(t >= 0, t, 0.01 * t)
    h2 = jnp.dot(t, w_ref[...], preferred_element_type=jnp.float32) + b2_ref[...]
    h2 = jnp.where(h2 >= 0, h2, 0.01 * h2)
    if chunked:
        for c in range(4):
            out_ref[c] = h2[:, c * _LANE:(c + 1) * _LANE]
    else:
        out_ref[...] = h2


def _tc2(h1, stats, g, be, w2, b2, chunked):
    if chunked:
        out_spec = pl.BlockSpec((4, _R, _LANE), lambda i: (0, i, 0))
        out_shape = jax.ShapeDtypeStruct((4, _N, _LANE), jnp.float32)
    else:
        out_spec = pl.BlockSpec((_R, 512), lambda i: (i, 0))
        out_shape = jax.ShapeDtypeStruct((_N, 512), jnp.float32)
    return pl.pallas_call(
        functools.partial(_tc2_body, chunked),
        grid=(_GRID,),
        in_specs=[
            pl.BlockSpec((_R, 512), lambda i: (i, 0)),
            pl.BlockSpec((8, 512), lambda i: (0, 0)),
            pl.BlockSpec((1, 512), lambda i: (0, 0)),
            pl.BlockSpec((1, 512), lambda i: (0, 0)),
            pl.BlockSpec((512, 512), lambda i: (0, 0)),
            pl.BlockSpec((1, 512), lambda i: (0, 0)),
        ],
        out_specs=out_spec,
        out_shape=out_shape,
    )(h1, stats, g, be, w2, b2)


def _tc3_body(h_ref, b_ref, fw_ref, fb_ref, lw_ref, lb_ref, out_ref, acc_ref):
    i = pl.program_id(0)

    @pl.when(i == 0)
    def _():
        acc_ref[...] = jnp.zeros((64, 512), jnp.float32)

    bvec = b_ref[0]
    gid = lax.broadcasted_iota(jnp.int32, (64, _R), 0)
    onehot = (bvec == gid).astype(jnp.float32)
    acc_ref[...] += jnp.dot(onehot, h_ref[...], preferred_element_type=jnp.float32)

    @pl.when(i == pl.num_programs(0) - 1)
    def _():
        t = jnp.dot(acc_ref[...], fw_ref[...],
                    preferred_element_type=jnp.float32) + fb_ref[...]
        t = jnp.where(t >= 0, t, 0.01 * t)
        out_ref[...] = jnp.dot(t, lw_ref[...],
                               preferred_element_type=jnp.float32) + lb_ref[...]


def _tc3(h, batch3, ffn_W, ffn_b, final_W, final_b):
    return pl.pallas_call(
        _tc3_body,
        grid=(_GRID,),
        in_specs=[
            pl.BlockSpec((_R, 512), lambda i: (i, 0)),
            pl.BlockSpec((1, 1, _R), lambda i: (i, 0, 0)),
            pl.BlockSpec((512, 512), lambda i: (0, 0)),
            pl.BlockSpec((1, 512), lambda i: (0, 0)),
            pl.BlockSpec((512, 1), lambda i: (0, 0)),
            pl.BlockSpec((1, 1), lambda i: (0, 0)),
        ],
        out_specs=pl.BlockSpec((64, 1), lambda i: (0, 0)),
        out_shape=jax.ShapeDtypeStruct((64, 1), jnp.float32),
        scratch_shapes=[pltpu.VMEM((64, 512), jnp.float32)],
    )(h, batch3, ffn_W, ffn_b, final_W, final_b)


_make_sc_segsum = functools.lru_cache(maxsize=None)(_make_sc_segsum)


def kernel(x, edge_index, batch,
           conv0_W1, conv0_b1, conv0_g, conv0_be, conv0_W2, conv0_b2,
           conv1_W1, conv1_b1, conv1_g, conv1_be, conv1_W2, conv1_b2,
           conv2_W1, conv2_b1, conv2_g, conv2_be, conv2_W2, conv2_b2,
           ffn_W, ffn_b, final_W, final_b):
    x = x.astype(jnp.float32)
    src3 = edge_index[0].reshape(_NW, _NBLK, _B)
    dst3 = edge_index[1].reshape(_NW, _NBLK, _B)
    zeros = jnp.zeros((_P, _LANE), jnp.float32)
    batch3 = batch.reshape(_GRID, 1, _R)

    x_ch = x.reshape(_N, 2, _LANE).transpose(1, 0, 2)
    layers = [
        (2, conv0_W1.reshape(2, _LANE, 512), conv0_b1.reshape(1, 512),
         conv0_g.reshape(1, 512), conv0_be.reshape(1, 512), conv0_W2,
         conv0_b2.reshape(1, 512)),
        (4, conv1_W1.reshape(4, _LANE, 512), conv1_b1.reshape(1, 512),
         conv1_g.reshape(1, 512), conv1_be.reshape(1, 512), conv1_W2,
         conv1_b2.reshape(1, 512)),
        (4, conv2_W1.reshape(4, _LANE, 512), conv2_b1.reshape(1, 512),
         conv2_g.reshape(1, 512), conv2_be.reshape(1, 512), conv2_W2,
         conv2_b2.reshape(1, 512)),
    ]
    h_ch = x_ch
    for li, (cch, w1c, b1, g, be, w2, b2) in enumerate(layers):
        agg2 = _make_sc_segsum(cch)(h_ch, src3, dst3, zeros)
        h1, stats = _tc1(h_ch, agg2, w1c, b1, cch)
        last = li == 2
        h_ch = _tc2(h1, stats, g, be, w2, b2, chunked=not last)

    out = _tc3(h_ch, batch3, ffn_W, ffn_b.reshape(1, 512),
               final_W, final_b.reshape(1, 1))
    return out


# speedup vs baseline: 1.2967x; 1.2967x over previous
import functools

import jax
import jax.numpy as jnp
from jax import lax
from jax.experimental import pallas as pl
from jax.experimental.pallas import tpu as pltpu
from jax.experimental.pallas import tpu_sc as plsc

_N = 10000
_E = 160000
_LANE = 128
_NSUB = 16
_NCORE = 2
_NW = _NCORE * _NSUB
_EPW = _E // _NW
_B = 125
_NBLK = _EPW // _B
_NPAD = 10240
_RPS = _NPAD // _NSUB
_P = 640
_NPIECE = _RPS // _P
_PRB = 80
_NPRB = _RPS // _PRB
_R = 1000
_GRID = _N // _R


def _make_sc_segsum(c_chunks):
    mesh = plsc.VectorSubcoreMesh(core_axis_name="c", subcore_axis_name="s")

    @functools.partial(
        pl.kernel,
        out_type=jax.ShapeDtypeStruct((_NCORE, c_chunks, _NPAD, _LANE), jnp.float32),
        mesh=mesh,
        scratch_types=[
            pltpu.VMEM((_NBLK, _B), jnp.int32),
            pltpu.VMEM((_NBLK, _B), jnp.int32),
            pltpu.VMEM((_B, _LANE), jnp.float32),
            pltpu.VMEM((_B, _LANE), jnp.float32),
            pltpu.SemaphoreType.DMA,
            pltpu.SemaphoreType.DMA,
            pltpu.VMEM_SHARED((_NPAD, _LANE), jnp.float32),
        ],
    )
    def segsum(x_hbm, src_hbm, dst_hbm, z_hbm, out_hbm,
               src_v, dst_v, b0, b1, g0, g1, acc_sh):
        core = lax.axis_index("c")
        sub = lax.axis_index("s")
        wid = core * _NSUB + sub
        bufs = (b0, b1)
        gs = (g0, g1)
        pltpu.sync_copy(src_hbm.at[wid], src_v)
        pltpu.sync_copy(dst_hbm.at[wid], dst_v)

        def gather(c, k, i):
            return pltpu.make_async_copy(x_hbm.at[c].at[src_v.at[k]], bufs[i],
                                         gs[i])

        for c in range(c_chunks):
            for p in range(_NPIECE):
                pltpu.sync_copy(z_hbm, acc_sh.at[pl.ds(sub * _RPS + p * _P, _P)])
            plsc.subcore_barrier()

            for i in range(2):
                gather(c, i, i).start()

            @pl.loop(0, _NBLK - 2, step=2)
            def _(j):
                for i in range(2):
                    k = j + i
                    gather(c, k, i).wait()
                    pltpu.sync_copy(bufs[i], acc_sh.at[dst_v.at[k]], add=True)
                    gather(c, k + 2, i).start()

            for k in range(_NBLK - 2, _NBLK):
                i = k % 2
                gather(c, k, i).wait()
                pltpu.sync_copy(bufs[i], acc_sh.at[dst_v.at[k]], add=True)

            plsc.subcore_barrier()
            for p in range(_NPRB):
                r0 = sub * _RPS + p * _PRB
                pltpu.sync_copy(acc_sh.at[pl.ds(r0, _PRB)], b0.at[pl.ds(0, _PRB)])
                pltpu.sync_copy(b0.at[pl.ds(0, _PRB)], out_hbm.at[core, c, pl.ds(r0, _PRB)])
            plsc.subcore_barrier()

    return segsum


def _tc1_body(c_chunks, x_ref, a_ref, w_ref, b1_ref, h1_ref, st_ref):
    acc = jnp.zeros((_R, 512), jnp.float32)
    for c in range(c_chunks):
        xa = x_ref[c] + a_ref[0, c] + a_ref[1, c]
        acc += jnp.dot(xa, w_ref[c], preferred_element_type=jnp.float32)
    h1 = acc + b1_ref[...]
    h1_ref[...] = h1
    i = pl.program_id(0)

    @pl.when(i == 0)
    def _():
        st_ref[...] = jnp.zeros((8, 512), jnp.float32)

    st_ref[0:1, :] += jnp.sum(h1, axis=0, keepdims=True)
    st_ref[1:2, :] += jnp.sum(h1 * h1, axis=0, keepdims=True)


def _tc1(x_ch, agg2, w1c, b1, c_chunks):
    return pl.pallas_call(
        functools.partial(_tc1_body, c_chunks),
        grid=(_GRID,),
        in_specs=[
            pl.BlockSpec((c_chunks, _R, _LANE), lambda i: (0, i, 0)),
            pl.BlockSpec((_NCORE, c_chunks, _R, _LANE), lambda i: (0, 0, i, 0)),
            pl.BlockSpec((c_chunks, _LANE, 512), lambda i: (0, 0, 0)),
            pl.BlockSpec((1, 512), lambda i: (0, 0)),
        ],
        out_specs=[
            pl.BlockSpec((_R, 512), lambda i: (i, 0)),
            pl.BlockSpec((8, 512), lambda i: (0, 0)),
        ],
        out_shape=[
            jax.ShapeDtypeStruct((_N, 512), jnp.float32),
            jax.ShapeDtypeStruct((8, 512), jnp.float32),
        ],
    )(x_ch, agg2, w1c, b1)


def _tc2_body(chunked, h1_ref, st_ref, g_ref, be_ref, w_ref, b2_ref, out_ref):
    inv_n = 1.0 / _N
    mu = st_ref[0:1, :] * inv_n
    ex2 = st_ref[1:2, :] * inv_n
    var = ex2 - mu * mu
    a = g_ref[...] * lax.rsqrt(var + 1e-5)
    cc = be_ref[...] - mu * a
    t = h1_ref[...] * a + cc
    t = jnp.where(t >= 0, t, 0.01 * t)
    h2 = jnp.dot(t, w_ref[...], preferred_element_type=jnp.float32) + b2_ref[...]
    h2 = jnp.where(h2 >= 0, h2, 0.01 * h2)
    if chunked:
        for c in range(4):
            out_ref[c] = h2[:, c * _LANE:(c + 1) * _LANE]
    else:
        out_ref[...] = h2


def _tc2(h1, stats, g, be, w2, b2, chunked):
    if chunked:
        out_spec = pl.BlockSpec((4, _R, _LANE), lambda i: (0, i, 0))
        out_shape = jax.ShapeDtypeStruct((4, _N, _LANE), jnp.float32)
    else:
        out_spec = pl.BlockSpec((_R, 512), lambda i: (i, 0))
        out_shape = jax.ShapeDtypeStruct((_N, 512), jnp.float32)
    return pl.pallas_call(
        functools.partial(_tc2_body, chunked),
        grid=(_GRID,),
        in_specs=[
            pl.BlockSpec((_R, 512), lambda i: (i, 0)),
            pl.BlockSpec((8, 512), lambda i: (0, 0)),
            pl.BlockSpec((1, 512), lambda i: (0, 0)),
            pl.BlockSpec((1, 512), lambda i: (0, 0)),
            pl.BlockSpec((512, 512), lambda i: (0, 0)),
            pl.BlockSpec((1, 512), lambda i: (0, 0)),
        ],
        out_specs=out_spec,
        out_shape=out_shape,
    )(h1, stats, g, be, w2, b2)


def _tc3_body(h_ref, b_ref, fw_ref, fb_ref, lw_ref, lb_ref, out_ref, acc_ref):
    i = pl.program_id(0)

    @pl.when(i == 0)
    def _():
        acc_ref[...] = jnp.zeros((64, 512), jnp.float32)

    bvec = b_ref[0]
    gid = lax.broadcasted_iota(jnp.int32, (64, _R), 0)
    onehot = (bvec == gid).astype(jnp.float32)
    acc_ref[...] += jnp.dot(onehot, h_ref[...], preferred_element_type=jnp.float32)

    @pl.when(i == pl.num_programs(0) - 1)
    def _():
        t = jnp.dot(acc_ref[...], fw_ref[...],
                    preferred_element_type=jnp.float32) + fb_ref[...]
        t = jnp.where(t >= 0, t, 0.01 * t)
        out_ref[...] = jnp.dot(t, lw_ref[...],
                               preferred_element_type=jnp.float32) + lb_ref[...]


def _tc3(h, batch3, ffn_W, ffn_b, final_W, final_b):
    return pl.pallas_call(
        _tc3_body,
        grid=(_GRID,),
        in_specs=[
            pl.BlockSpec((_R, 512), lambda i: (i, 0)),
            pl.BlockSpec((1, 1, _R), lambda i: (i, 0, 0)),
            pl.BlockSpec((512, 512), lambda i: (0, 0)),
            pl.BlockSpec((1, 512), lambda i: (0, 0)),
            pl.BlockSpec((512, 1), lambda i: (0, 0)),
            pl.BlockSpec((1, 1), lambda i: (0, 0)),
        ],
        out_specs=pl.BlockSpec((64, 1), lambda i: (0, 0)),
        out_shape=jax.ShapeDtypeStruct((64, 1), jnp.float32),
        scratch_shapes=[pltpu.VMEM((64, 512), jnp.float32)],
    )(h, batch3, ffn_W, ffn_b, final_W, final_b)


_make_sc_segsum = functools.lru_cache(maxsize=None)(_make_sc_segsum)


def kernel(x, edge_index, batch,
           conv0_W1, conv0_b1, conv0_g, conv0_be, conv0_W2, conv0_b2,
           conv1_W1, conv1_b1, conv1_g, conv1_be, conv1_W2, conv1_b2,
           conv2_W1, conv2_b1, conv2_g, conv2_be, conv2_W2, conv2_b2,
           ffn_W, ffn_b, final_W, final_b):
    x = x.astype(jnp.float32)
    src3 = edge_index[0].reshape(_NW, _NBLK, _B)
    dst3 = edge_index[1].reshape(_NW, _NBLK, _B)
    zeros = jnp.zeros((_P, _LANE), jnp.float32)
    batch3 = batch.reshape(_GRID, 1, _R)

    x_ch = x.reshape(_N, 2, _LANE).transpose(1, 0, 2)
    layers = [
        (2, conv0_W1.reshape(2, _LANE, 512), conv0_b1.reshape(1, 512),
         conv0_g.reshape(1, 512), conv0_be.reshape(1, 512), conv0_W2,
         conv0_b2.reshape(1, 512)),
        (4, conv1_W1.reshape(4, _LANE, 512), conv1_b1.reshape(1, 512),
         conv1_g.reshape(1, 512), conv1_be.reshape(1, 512), conv1_W2,
         conv1_b2.reshape(1, 512)),
        (4, conv2_W1.reshape(4, _LANE, 512), conv2_b1.reshape(1, 512),
         conv2_g.reshape(1, 512), conv2_be.reshape(1, 512), conv2_W2,
         conv2_b2.reshape(1, 512)),
    ]
    h_ch = x_ch
    for li, (cch, w1c, b1, g, be, w2, b2) in enumerate(layers):
        agg2 = _make_sc_segsum(cch)(h_ch, src3, dst3, zeros)
        h1, stats = _tc1(h_ch, agg2, w1c, b1, cch)
        last = li == 2
        h_ch = _tc2(h1, stats, g, be, w2, b2, chunked=not last)

    out = _tc3(h_ch, batch3, ffn_W, ffn_b.reshape(1, 512),
               final_W, final_b.reshape(1, 1))
    return out


# speedup vs baseline: 1.3122x; 1.0120x over previous
import functools

import jax
import jax.numpy as jnp
from jax import lax
from jax.experimental import pallas as pl
from jax.experimental.pallas import tpu as pltpu
from jax.experimental.pallas import tpu_sc as plsc

_N = 10000
_E = 160000
_LANE = 128
_NSUB = 16
_NCORE = 2
_NW = _NCORE * _NSUB
_EPW = _E // _NW
_B = 125
_NBLK = _EPW // _B
_NPAD = 10240
_RPS = _NPAD // _NSUB
_P = 640
_NPIECE = _RPS // _P
_PRB = 80
_NPRB = _RPS // _PRB
_R = 1000
_GRID = _N // _R


def _make_sc_segsum(c_chunks):
    mesh = plsc.VectorSubcoreMesh(core_axis_name="c", subcore_axis_name="s")

    @functools.partial(
        pl.kernel,
        out_type=jax.ShapeDtypeStruct((_NCORE, c_chunks, _NPAD, _LANE), jnp.float32),
        mesh=mesh,
        scratch_types=[
            pltpu.VMEM((_NBLK, _B), jnp.int32),
            pltpu.VMEM((_NBLK, _B), jnp.int32),
            pltpu.VMEM((_B, _LANE), jnp.float32),
            pltpu.VMEM((_B, _LANE), jnp.float32),
            pltpu.SemaphoreType.DMA,
            pltpu.SemaphoreType.DMA,
            pltpu.VMEM_SHARED((_NPAD, _LANE), jnp.float32),
        ],
    )
    def segsum(x_hbm, src_hbm, dst_hbm, z_hbm, out_hbm,
               src_v, dst_v, b0, b1, g0, g1, acc_sh):
        core = lax.axis_index("c")
        sub = lax.axis_index("s")
        wid = core * _NSUB + sub
        bufs = (b0, b1)
        gs = (g0, g1)
        pltpu.sync_copy(src_hbm.at[wid], src_v)
        pltpu.sync_copy(dst_hbm.at[wid], dst_v)

        def gather(c, k, i):
            return pltpu.make_async_copy(x_hbm.at[c].at[src_v.at[k]], bufs[i],
                                         gs[i])

        for c in range(c_chunks):
            for p in range(_NPIECE):
                pltpu.sync_copy(z_hbm, acc_sh.at[pl.ds(sub * _RPS + p * _P, _P)])
            plsc.subcore_barrier()

            for i in range(2):
                gather(c, i, i).start()

            @pl.loop(0, _NBLK - 2, step=2)
            def _(j):
                for i in range(2):
                    k = j + i
                    gather(c, k, i).wait()
                    pltpu.sync_copy(bufs[i], acc_sh.at[dst_v.at[k]], add=True)
                    gather(c, k + 2, i).start()

            for k in range(_NBLK - 2, _NBLK):
                i = k % 2
                gather(c, k, i).wait()
                pltpu.sync_copy(bufs[i], acc_sh.at[dst_v.at[k]], add=True)

            plsc.subcore_barrier()
            r0 = sub * _RPS
            pltpu.sync_copy(acc_sh.at[pl.ds(r0, _RPS)],
                            out_hbm.at[core, c, pl.ds(r0, _RPS)])
            plsc.subcore_barrier()

    return segsum


def _tc1_body(c_chunks, x_ref, a_ref, w_ref, b1_ref, h1_ref, st_ref):
    acc = jnp.zeros((_R, 512), jnp.float32)
    for c in range(c_chunks):
        xa = x_ref[c] + a_ref[0, c] + a_ref[1, c]
        acc += jnp.dot(xa, w_ref[c], preferred_element_type=jnp.float32)
    h1 = acc + b1_ref[...]
    h1_ref[...] = h1
    i = pl.program_id(0)

    @pl.when(i == 0)
    def _():
        st_ref[...] = jnp.zeros((8, 512), jnp.float32)

    st_ref[0:1, :] += jnp.sum(h1, axis=0, keepdims=True)
    st_ref[1:2, :] += jnp.sum(h1 * h1, axis=0, keepdims=True)


def _tc1(x_ch, agg2, w1c, b1, c_chunks):
    return pl.pallas_call(
        functools.partial(_tc1_body, c_chunks),
        grid=(_GRID,),
        in_specs=[
            pl.BlockSpec((c_chunks, _R, _LANE), lambda i: (0, i, 0)),
            pl.BlockSpec((_NCORE, c_chunks, _R, _LANE), lambda i: (0, 0, i, 0)),
            pl.BlockSpec((c_chunks, _LANE, 512), lambda i: (0, 0, 0)),
            pl.BlockSpec((1, 512), lambda i: (0, 0)),
        ],
        out_specs=[
            pl.BlockSpec((_R, 512), lambda i: (i, 0)),
            pl.BlockSpec((8, 512), lambda i: (0, 0)),
        ],
        out_shape=[
            jax.ShapeDtypeStruct((_N, 512), jnp.float32),
            jax.ShapeDtypeStruct((8, 512), jnp.float32),
        ],
    )(x_ch, agg2, w1c, b1)


def _tc2_body(chunked, h1_ref, st_ref, g_ref, be_ref, w_ref, b2_ref, out_ref):
    inv_n = 1.0 / _N
    mu = st_ref[0:1, :] * inv_n
    ex2 = st_ref[1:2, :] * inv_n
    var = ex2 - mu * mu
    a = g_ref[...] * lax.rsqrt(var + 1e-5)
    cc = be_ref[...] - mu * a
    t = h1_ref[...] * a + cc
    t = jnp.where(t >= 0, t, 0.01 * t)
    h2 = jnp.dot(t, w_ref[...], preferred_element_type=jnp.float32) + b2_ref[...]
    h2 = jnp.where(h2 >= 0, h2, 0.01 * h2)
    if chunked:
        for c in range(4):
            out_ref[c] = h2[:, c * _LANE:(c + 1) * _LANE]
    else:
        out_ref[...] = h2


def _tc2(h1, stats, g, be, w2, b2, chunked):
    if chunked:
        out_spec = pl.BlockSpec((4, _R, _LANE), lambda i: (0, i, 0))
        out_shape = jax.ShapeDtypeStruct((4, _N, _LANE), jnp.float32)
    else:
        out_spec = pl.BlockSpec((_R, 512), lambda i: (i, 0))
        out_shape = jax.ShapeDtypeStruct((_N, 512), jnp.float32)
    return pl.pallas_call(
        functools.partial(_tc2_body, chunked),
        grid=(_GRID,),
        in_specs=[
            pl.BlockSpec((_R, 512), lambda i: (i, 0)),
            pl.BlockSpec((8, 512), lambda i: (0, 0)),
            pl.BlockSpec((1, 512), lambda i: (0, 0)),
            pl.BlockSpec((1, 512), lambda i: (0, 0)),
            pl.BlockSpec((512, 512), lambda i: (0, 0)),
            pl.BlockSpec((1, 512), lambda i: (0, 0)),
        ],
        out_specs=out_spec,
        out_shape=out_shape,
    )(h1, stats, g, be, w2, b2)


def _tc3_body(h_ref, b_ref, fw_ref, fb_ref, lw_ref, lb_ref, out_ref, acc_ref):
    i = pl.program_id(0)

    @pl.when(i == 0)
    def _():
        acc_ref[...] = jnp.zeros((64, 512), jnp.float32)

    bvec = b_ref[0]
    gid = lax.broadcasted_iota(jnp.int32, (64, _R), 0)
    onehot = (bvec == gid).astype(jnp.float32)
    acc_ref[...] += jnp.dot(onehot, h_ref[...], preferred_element_type=jnp.float32)

    @pl.when(i == pl.num_programs(0) - 1)
    def _():
        t = jnp.dot(acc_ref[...], fw_ref[...],
                    preferred_element_type=jnp.float32) + fb_ref[...]
        t = jnp.where(t >= 0, t, 0.01 * t)
        out_ref[...] = jnp.dot(t, lw_ref[...],
                               preferred_element_type=jnp.float32) + lb_ref[...]


def _tc3(h, batch3, ffn_W, ffn_b, final_W, final_b):
    return pl.pallas_call(
        _tc3_body,
        grid=(_GRID,),
        in_specs=[
            pl.BlockSpec((_R, 512), lambda i: (i, 0)),
            pl.BlockSpec((1, 1, _R), lambda i: (i, 0, 0)),
            pl.BlockSpec((512, 512), lambda i: (0, 0)),
            pl.BlockSpec((1, 512), lambda i: (0, 0)),
            pl.BlockSpec((512, 1), lambda i: (0, 0)),
            pl.BlockSpec((1, 1), lambda i: (0, 0)),
        ],
        out_specs=pl.BlockSpec((64, 1), lambda i: (0, 0)),
        out_shape=jax.ShapeDtypeStruct((64, 1), jnp.float32),
        scratch_shapes=[pltpu.VMEM((64, 512), jnp.float32)],
    )(h, batch3, ffn_W, ffn_b, final_W, final_b)


_make_sc_segsum = functools.lru_cache(maxsize=None)(_make_sc_segsum)


def kernel(x, edge_index, batch,
           conv0_W1, conv0_b1, conv0_g, conv0_be, conv0_W2, conv0_b2,
           conv1_W1, conv1_b1, conv1_g, conv1_be, conv1_W2, conv1_b2,
           conv2_W1, conv2_b1, conv2_g, conv2_be, conv2_W2, conv2_b2,
           ffn_W, ffn_b, final_W, final_b):
    x = x.astype(jnp.float32)
    src3 = edge_index[0].reshape(_NW, _NBLK, _B)
    dst3 = edge_index[1].reshape(_NW, _NBLK, _B)
    zeros = jnp.zeros((_P, _LANE), jnp.float32)
    batch3 = batch.reshape(_GRID, 1, _R)

    x_ch = x.reshape(_N, 2, _LANE).transpose(1, 0, 2)
    layers = [
        (2, conv0_W1.reshape(2, _LANE, 512), conv0_b1.reshape(1, 512),
         conv0_g.reshape(1, 512), conv0_be.reshape(1, 512), conv0_W2,
         conv0_b2.reshape(1, 512)),
        (4, conv1_W1.reshape(4, _LANE, 512), conv1_b1.reshape(1, 512),
         conv1_g.reshape(1, 512), conv1_be.reshape(1, 512), conv1_W2,
         conv1_b2.reshape(1, 512)),
        (4, conv2_W1.reshape(4, _LANE, 512), conv2_b1.reshape(1, 512),
         conv2_g.reshape(1, 512), conv2_be.reshape(1, 512), conv2_W2,
         conv2_b2.reshape(1, 512)),
    ]
    h_ch = x_ch
    for li, (cch, w1c, b1, g, be, w2, b2) in enumerate(layers):
        agg2 = _make_sc_segsum(cch)(h_ch, src3, dst3, zeros)
        h1, stats = _tc1(h_ch, agg2, w1c, b1, cch)
        last = li == 2
        h_ch = _tc2(h1, stats, g, be, w2, b2, chunked=not last)

    out = _tc3(h_ch, batch3, ffn_W, ffn_b.reshape(1, 512),
               final_W, final_b.reshape(1, 1))
    return out
